# baseline (speedup 1.0000x reference)
import functools

import jax
import jax.numpy as jnp
from jax import lax
from jax.experimental import pallas as pl
from jax.experimental.pallas import tpu as pltpu
from jax.experimental.pallas import tpu_sc as plsc

N = 10000
NP = 10240
E = 320000
EP = 321536
R = 128
H = 768
L = 8
VOCAB = 32000

_INFO = plsc.get_sparse_core_info()
NC = _INFO.num_cores
NS = _INFO.num_subcores
NWORK = NC * NS
NODES_W = NP // NWORK
NTASK = 64
NPT = NP // NTASK
SCAN = 2048
NCH = EP // SCAN
NCH_PAD = 160
GC = 128
BIG = 3.0e38

_MESH = plsc.VectorSubcoreMesh(core_axis_name="c", subcore_axis_name="s")
_SC_PARAMS = pltpu.CompilerParams(needs_layout_passes=False)


def _wid():
    return lax.axis_index("s") * NC + lax.axis_index("c")


def _vextract(ref, j):
    base = lax.shift_left(lax.shift_right_logical(j, 4), 4)
    vec = ref[pl.ds(base, 16)]
    lane = lax.iota(jnp.int32, 16)
    return jnp.sum(jnp.where(lane == (j - base), vec, 0))


def _splat(x):
    return jnp.zeros((16,), jnp.int32) + x


def _proj_body(emb_ref, w_ref, o_ref):
    i = pl.program_id(0)
    acc = jnp.dot(emb_ref[...], w_ref[...], preferred_element_type=jnp.float32)
    rows = lax.broadcasted_iota(jnp.int32, acc.shape, 0)
    o_ref[...] = jnp.where((rows == 0) & (i == 0), 0.0, acc)


def _proj_table(emb, w):
    tile = 2000
    return pl.pallas_call(
        _proj_body,
        grid=(VOCAB // tile,),
        in_specs=[
            pl.BlockSpec((tile, H), lambda i: (i, 0)),
            pl.BlockSpec((H, R), lambda i: (0, 0)),
        ],
        out_specs=pl.BlockSpec((tile, R), lambda i: (i, 0)),
        out_shape=jax.ShapeDtypeStruct((VOCAB, R), jnp.float32),
    )(emb, w)


@functools.partial(
    pl.kernel,
    mesh=_MESH,
    compiler_params=_SC_PARAMS,
    out_type=(
        jax.ShapeDtypeStruct((NP, R), jnp.float32),
        jax.ShapeDtypeStruct((NP, R), jnp.int32),
    ),
    scratch_types=[
        pltpu.VMEM((NODES_W,), jnp.int32),
        pltpu.VMEM((NODES_W, R), jnp.int32),
        pltpu.VMEM((40, 128), jnp.int32),
        pltpu.VMEM((128, R), jnp.float32),
        pltpu.VMEM((8, R), jnp.float32),
        pltpu.SemaphoreType.DMA,
    ],
)
def _ctx_gather(cand_hbm, kglp_hbm, proj_hbm, xsum_hbm, tokout_hbm,
                cand_v, tok_v, idx_v, rows_v, acc_v, sem):
    base = _wid() * NODES_W
    pltpu.sync_copy(cand_hbm.at[pl.ds(base, NODES_W)], cand_v)
    pltpu.async_copy(kglp_hbm.at[cand_v], tok_v, sem).wait()
    pltpu.sync_copy(tok_v, tokout_hbm.at[pl.ds(base, NODES_W)])

    for n in range(NODES_W):
        idx_v[n // 8, pl.ds((n % 8) * 16, 16)] = tok_v[n, pl.ds(0, 16)]

    def chunk(cch, _):
        pltpu.async_copy(proj_hbm.at[idx_v.at[cch]], rows_v, sem).wait()
        for j in range(8):
            for v in range(R // 16):
                sl = pl.ds(v * 16, 16)
                s = rows_v[j * 16, sl]
                for l in range(1, 16):
                    s = s + rows_v[j * 16 + l, sl]
                acc_v[j, sl] = s
        pltpu.sync_copy(acc_v, xsum_hbm.at[pl.ds(base + cch * 8, 8)])
        return 0

    lax.fori_loop(0, 40, chunk, 0)


def _xfin_body(xs_ref, tok_ref, db_ref, o_ref):
    cnt = jnp.sum((tok_ref[...] != 0).astype(jnp.float32), axis=1, keepdims=True)
    x = xs_ref[...] / jnp.maximum(cnt, 1.0)
    o_ref[...] = x + jnp.where(cnt > 0.0, db_ref[...], 0.0)


def _x_finalize(xsum, tok, down_b):
    tile = 1024
    return pl.pallas_call(
        _xfin_body,
        grid=(NP // tile,),
        in_specs=[
            pl.BlockSpec((tile, R), lambda i: (i, 0)),
            pl.BlockSpec((tile, R), lambda i: (i, 0)),
            pl.BlockSpec((1, R), lambda i: (0, 0)),
        ],
        out_specs=pl.BlockSpec((tile, R), lambda i: (i, 0)),
        out_shape=jax.ShapeDtypeStruct((NP, R), jnp.float32),
    )(xsum, tok, down_b)


def _gate_body(x_ref, lh_ref, qw_ref, gwx_ref, gwq_ref, gb_ref, w1_ref, b1_ref,
               w2r_ref, b2_ref, pwt_ref, pwb_ref, xg_ref, a_ref, b_ref):
    q = jnp.dot(lh_ref[...], qw_ref[...], preferred_element_type=jnp.float32)
    x = x_ref[...]
    gi = jnp.dot(x, gwx_ref[...], preferred_element_type=jnp.float32)
    gi = gi + jnp.dot(q, gwq_ref[...], preferred_element_type=jnp.float32)
    gi = jnp.maximum(gi + gb_ref[...], 0.0)
    hm = jnp.maximum(
        jnp.dot(gi, w1_ref[...], preferred_element_type=jnp.float32) + b1_ref[...], 0.0)
    gl = jnp.sum(hm * w2r_ref[...], axis=1, keepdims=True) + b2_ref[0, 0]
    gate = 1.0 / (1.0 + jnp.exp(-gl))
    xg = x * gate
    xg_ref[...] = xg
    a_ref[...] = jnp.dot(xg, pwt_ref[...], preferred_element_type=jnp.float32)
    b_ref[...] = jnp.dot(xg, pwb_ref[...], preferred_element_type=jnp.float32)


def _gate_stage(x, lh, qw, gwx, gwq, gb, w1, b1, w2r, b2, pwt, pwb):
    tile = 1024
    full = lambda r, c: pl.BlockSpec((r, c), lambda i: (0, 0))
    nod = pl.BlockSpec((tile, R), lambda i: (i, 0))
    return pl.pallas_call(
        _gate_body,
        grid=(NP // tile,),
        in_specs=[nod, full(1, H), full(H, R), full(R, R), full(R, R),
                  full(1, R), full(R, R), full(1, R), full(1, R), full(1, R),
                  full(R, R), full(R, R)],
        out_specs=[nod, nod, nod],
        out_shape=[jax.ShapeDtypeStruct((NP, R), jnp.float32)] * 3,
    )(x, lh, qw, gwx, gwq, gb, w1, b1, w2r, b2, pwt, pwb)


@functools.partial(
    pl.kernel,
    mesh=_MESH,
    compiler_params=_SC_PARAMS,
    out_type=(
        jax.ShapeDtypeStruct((NTASK, NCH, SCAN), jnp.int32),
        jax.ShapeDtypeStruct((NTASK, NCH_PAD), jnp.int32),
    ),
    scratch_types=[
        pltpu.VMEM((SCAN,), jnp.int32),
        pltpu.VMEM((SCAN,), jnp.int32),
        pltpu.VMEM((SCAN + 16,), jnp.int32),
        pltpu.VMEM((NCH_PAD,), jnp.int32),
    ],
)
def _bin_edges(src_hbm, dst_hbm, bins_hbm, counts_hbm, srcv, dstv, stagev, cntv):
    w = _wid()
    lane = lax.iota(jnp.int32, 16)
    for tt in range(2):
        t = w * 2 + tt
        lo = t * NPT

        for j in range(NCH_PAD // 16):
            cntv[pl.ds(j * 16, 16)] = jnp.zeros((16,), jnp.int32)

        def chunk_body(c, _):
            pltpu.sync_copy(src_hbm.at[c], srcv)
            pltpu.sync_copy(dst_hbm.at[c], dstv)
            cnt = jnp.int32(0)
            for i in range(SCAN // 16):
                sl = pl.ds(i * 16, 16)
                d = dstv[sl]
                s = srcv[sl]
                m = (d >= lo) & (d < lo + NPT)
                pk = (s << 8) | (d - lo)
                mi = m.astype(jnp.int32)
                cs = plsc.cumsum(mi)
                pos = cnt + cs - mi
                plsc.store_scatter(stagev, [pos], pk, mask=m)
                cnt = cnt + cs[15]
            pltpu.sync_copy(stagev.at[pl.ds(0, SCAN)], bins_hbm.at[t, c])
            plsc.store_scatter(cntv, [_splat(c)], _splat(cnt), mask=(lane == 0))
            return 0

        lax.fori_loop(0, NCH, chunk_body, 0)
        pltpu.sync_copy(cntv, counts_hbm.at[t])


def _make_seg(compute_deg):
    outs = [
        jax.ShapeDtypeStruct((NP, R), jnp.float32),
        jax.ShapeDtypeStruct((NP, R), jnp.float32),
        jax.ShapeDtypeStruct((NP, R), jnp.float32),
        jax.ShapeDtypeStruct((NP, R), jnp.float32),
    ]
    scr = [
        pltpu.VMEM((NPT, R), jnp.float32),
        pltpu.VMEM((NPT, R), jnp.float32),
        pltpu.VMEM((NPT, R), jnp.float32),
        pltpu.VMEM((NPT, R), jnp.float32),
        pltpu.VMEM((GC, R), jnp.float32),
        pltpu.VMEM((GC,), jnp.int32),
        pltpu.VMEM((GC,), jnp.int32),
        pltpu.VMEM((NCH_PAD,), jnp.int32),
        pltpu.SemaphoreType.DMA,
    ]
    if compute_deg:
        outs.append(jax.ShapeDtypeStruct((NP, 16), jnp.float32))
        scr.insert(-1, pltpu.VMEM((NPT, 16), jnp.float32))

    @functools.partial(pl.kernel, mesh=_MESH, out_type=tuple(outs),
                       compiler_params=_SC_PARAMS, scratch_types=scr)
    def _seg(bins_hbm, counts_hbm, b_hbm, *refs):
        if compute_deg:
            (ss_hbm, sq_hbm, mn_hbm, mx_hbm, deg_hbm,
             asum, asq, amn, amx, rowsv, pkv, idxv, cntrow, degv, sem) = refs
        else:
            (ss_hbm, sq_hbm, mn_hbm, mx_hbm,
             asum, asq, amn, amx, rowsv, pkv, idxv, cntrow, sem) = refs
        w = _wid()
        lane = lax.iota(jnp.int32, 16)
        zero = jnp.zeros((16,), jnp.float32)
        lanec = [lane + v * 16 for v in range(R // 16)]
        for tt in range(2):
            t = w * 2 + tt
            lo = t * NPT

            def zb(j, _):
                jv = _splat(j)
                for v in range(R // 16):
                    plsc.store_scatter(asum, [jv, lanec[v]], zero)
                    plsc.store_scatter(asq, [jv, lanec[v]], zero)
                    plsc.store_scatter(amn, [jv, lanec[v]], zero + BIG)
                    plsc.store_scatter(amx, [jv, lanec[v]], zero - BIG)
                if compute_deg:
                    plsc.store_scatter(degv, [jv, lane], zero)
                return 0

            lax.fori_loop(0, NPT, zb, 0)
            pltpu.sync_copy(counts_hbm.at[t], cntrow)

            def cbody(c, _):
                cnt_c = _vextract(cntrow, c)

                def sbody(sub, _):
                    pltpu.sync_copy(bins_hbm.at[t, c, pl.ds(sub * GC, GC)], pkv)
                    base_s = _splat(sub * GC)
                    cnt_s = _splat(cnt_c)
                    for i in range(GC // 16):
                        sl = pl.ds(i * 16, 16)
                        pk = pkv[sl]
                        pos = base_s + (lane + i * 16)
                        idxv[sl] = jnp.where(pos < cnt_s, pk >> 8, 0)
                    pltpu.async_copy(b_hbm.at[idxv], rowsv, sem).wait()
                    m = jnp.minimum(cnt_c - sub * GC, GC)

                    def ebody(j, _):
                        jv = _splat(j)
                        dv = plsc.load_gather(pkv, [jv]) & 255
                        for v in range(R // 16):
                            lc = lanec[v]
                            r = plsc.load_gather(rowsv, [jv, lc])
                            s0 = plsc.load_gather(asum, [dv, lc])
                            plsc.store_scatter(asum, [dv, lc], s0 + r)
                            q0 = plsc.load_gather(asq, [dv, lc])
                            plsc.store_scatter(asq, [dv, lc], q0 + r * r)
                            m0 = plsc.load_gather(amn, [dv, lc])
                            plsc.store_scatter(amn, [dv, lc], jnp.minimum(m0, r))
                            x0 = plsc.load_gather(amx, [dv, lc])
                            plsc.store_scatter(amx, [dv, lc], jnp.maximum(x0, r))
                        if compute_deg:
                            d0 = plsc.load_gather(degv, [dv, lane])
                            plsc.store_scatter(degv, [dv, lane], d0 + 1.0)
                        return 0

                    lax.fori_loop(0, m, ebody, 0)
                    return 0

                nsub = lax.shift_right_logical(cnt_c + (GC - 1), 7)
                lax.fori_loop(0, nsub, sbody, 0)
                return 0

            lax.fori_loop(0, NCH, cbody, 0)
            sl = pl.ds(lo, NPT)
            pltpu.sync_copy(asum, ss_hbm.at[sl])
            pltpu.sync_copy(asq, sq_hbm.at[sl])
            pltpu.sync_copy(amn, mn_hbm.at[sl])
            pltpu.sync_copy(amx, mx_hbm.at[sl])
            if compute_deg:
                pltpu.sync_copy(degv, deg_hbm.at[sl])

    return _seg


_seg_deg = _make_seg(True)
_seg_nodeg = _make_seg(False)


def _post_body(x_ref, xg_ref, a_ref, ss_ref, sq_ref, mn_ref, mx_ref, deg_ref,
               hist_ref, preb_ref, pw_ref, pb_ref, lw_ref, lb_ref, o_ref):
    hist = hist_ref[...]
    binsv = lax.broadcasted_iota(jnp.int32, (1, R), 1).astype(jnp.float32)
    avg_log = jnp.sum(jnp.log(binsv + 1.0) * hist) / jnp.sum(hist)
    deg = deg_ref[...][:, 0:1]
    degc = jnp.maximum(deg, 1.0)
    hase = deg > 0.0
    ab = a_ref[...] + preb_ref[...]
    ssum = ss_ref[...]
    mean = jnp.where(hase, ab, 0.0) + ssum / degc
    s2 = (deg * ab * ab + 2.0 * ab * ssum + sq_ref[...]) / degc
    std = jnp.sqrt(jnp.maximum(s2 - mean * mean, 0.0) + 1e-5)
    mn = jnp.where(hase, ab + mn_ref[...], 0.0)
    mx = jnp.where(hase, ab + mx_ref[...], 0.0)
    log_deg = jnp.log(degc + 1.0)
    s = log_deg / avg_log
    tt = avg_log / log_deg
    blocks = [xg_ref[...], mean, mn, mx, std, mean * s, mn * s, mx * s,
              std * s, mean * tt, mn * tt, mx * tt, std * tt]
    pw = pw_ref[...]
    out = pb_ref[...]
    for k in range(13):
        out = out + jnp.dot(blocks[k], pw[k * R:(k + 1) * R, :],
                            preferred_element_type=jnp.float32)
    out = jnp.dot(out, lw_ref[...], preferred_element_type=jnp.float32) + lb_ref[...]
    o_ref[...] = jnp.maximum(out + x_ref[...], 0.0)


def _post_stage(x, xg, a, ss, sq, mn, mx, deg, hist, preb, pw, pb, lw, lb):
    tile = 1024
    nod = pl.BlockSpec((tile, R), lambda i: (i, 0))
    full = lambda r, c: pl.BlockSpec((r, c), lambda i: (0, 0))
    return pl.pallas_call(
        _post_body,
        grid=(NP // tile,),
        in_specs=[nod, nod, nod, nod, nod, nod, nod,
                  pl.BlockSpec((tile, 16), lambda i: (i, 0)),
                  full(1, R), full(1, R), full(13 * R, R), full(1, R),
                  full(R, R), full(1, R)],
        out_specs=nod,
        out_shape=jax.ShapeDtypeStruct((NP, R), jnp.float32),
    )(x, xg, a, ss, sq, mn, mx, deg, hist, preb, pw, pb, lw, lb)


def _score_body(x_ref, sw_ref, sb_ref, o_ref):
    o_ref[...] = jnp.sum(x_ref[...] * sw_ref[...], axis=1) + sb_ref[0, 0]


def _score_stage(x, swr, sb):
    tile = 1024
    return pl.pallas_call(
        _score_body,
        grid=(NP // tile,),
        in_specs=[pl.BlockSpec((tile, R), lambda i: (i, 0)),
                  pl.BlockSpec((1, R), lambda i: (0, 0)),
                  pl.BlockSpec((1, R), lambda i: (0, 0))],
        out_specs=pl.BlockSpec((tile,), lambda i: (i,)),
        out_shape=jax.ShapeDtypeStruct((NP,), jnp.float32),
    )(x, swr, sb)


def kernel(llm_hidden_state, params, candidate_ids, edge_index, kgl2token_ids,
           deg_histogram):
    p = params
    f32 = jnp.float32

    proj = _proj_table(p["llm_emb"], p["down_w"])

    cand = jnp.pad(candidate_ids, (0, NP - N))
    kglp = jnp.pad(kgl2token_ids, ((0, 0), (0, R - L)))
    xsum, tokrows = _ctx_gather(cand, kglp, proj)
    x = _x_finalize(xsum, tokrows, p["down_b"].reshape(1, R))

    src = jnp.pad(edge_index[0], (0, EP - E)).reshape(NCH, SCAN)
    dst = jnp.pad(edge_index[1], (0, EP - E),
                  constant_values=1 << 29).reshape(NCH, SCAN)
    bins, counts = _bin_edges(src, dst)

    hist = jnp.zeros((1, R), f32).at[0, :deg_histogram.shape[0]].set(
        deg_histogram.astype(f32))
    lh = llm_hidden_state
    qw = p["query_w"]

    deg = None
    for i in range(2):
        gw = p[f"l{i}_gate_w"]
        xg, a, b = _gate_stage(
            x, lh, qw, gw[:R], gw[R:], p[f"l{i}_gate_b"].reshape(1, R),
            p[f"l{i}_gmlp_w1"], p[f"l{i}_gmlp_b1"].reshape(1, R),
            p[f"l{i}_gmlp_w2"].reshape(1, R),
            jnp.broadcast_to(p[f"l{i}_gmlp_b2"].reshape(1, 1), (1, R)),
            p[f"l{i}_pre_w"][:R], p[f"l{i}_pre_w"][R:])
        if i == 0:
            ss, sq, mn, mx, deg = _seg_deg(bins, counts, b)
        else:
            ss, sq, mn, mx = _seg_nodeg(bins, counts, b)
        x = _post_stage(
            x, xg, a, ss, sq, mn, mx, deg, hist,
            p[f"l{i}_pre_b"].reshape(1, R), p[f"l{i}_post_w"],
            p[f"l{i}_post_b"].reshape(1, R), p[f"l{i}_lin_w"],
            p[f"l{i}_lin_b"].reshape(1, R))

    logits = _score_stage(x, p["score_w"].reshape(1, R),
                          jnp.broadcast_to(p["score_b"].reshape(1, 1), (1, R)))
    return logits[:N]


# speedup vs baseline: 1.0002x; 1.0002x over previous
import functools

import jax
import jax.numpy as jnp
from jax import lax
from jax.experimental import pallas as pl
from jax.experimental.pallas import tpu as pltpu
from jax.experimental.pallas import tpu_sc as plsc

N = 10000
NP = 10240
E = 320000
EP = 321536
R = 128
H = 768
L = 8
VOCAB = 32000

_INFO = plsc.get_sparse_core_info()
NC = _INFO.num_cores
NS = _INFO.num_subcores
NWORK = NC * NS
NODES_W = NP // NWORK
NTASK = 64
NPT = NP // NTASK
SCAN = 2048
NCH = EP // SCAN
NCH_PAD = 160
GC = 128
BIG = 3.0e38

_MESH = plsc.VectorSubcoreMesh(core_axis_name="c", subcore_axis_name="s")
_SC_PARAMS = pltpu.CompilerParams(needs_layout_passes=False)


def _wid():
    return lax.axis_index("s") * NC + lax.axis_index("c")


def _vextract(ref, j):
    base = lax.shift_left(lax.shift_right_logical(j, 4), 4)
    vec = ref[pl.ds(base, 16)]
    lane = lax.iota(jnp.int32, 16)
    return jnp.sum(jnp.where(lane == (j - base), vec, 0))


def _splat(x):
    return jnp.zeros((16,), jnp.int32) + x


def _proj_body(emb_ref, w_ref, o_ref):
    i = pl.program_id(0)
    acc = jnp.dot(emb_ref[...], w_ref[...], preferred_element_type=jnp.float32)
    rows = lax.broadcasted_iota(jnp.int32, acc.shape, 0)
    o_ref[...] = jnp.where((rows == 0) & (i == 0), 0.0, acc)


def _proj_table(emb, w):
    tile = 2000
    return pl.pallas_call(
        _proj_body,
        grid=(VOCAB // tile,),
        in_specs=[
            pl.BlockSpec((tile, H), lambda i: (i, 0)),
            pl.BlockSpec((H, R), lambda i: (0, 0)),
        ],
        out_specs=pl.BlockSpec((tile, R), lambda i: (i, 0)),
        out_shape=jax.ShapeDtypeStruct((VOCAB, R), jnp.float32),
    )(emb, w)


@functools.partial(
    pl.kernel,
    mesh=_MESH,
    compiler_params=_SC_PARAMS,
    out_type=(
        jax.ShapeDtypeStruct((NP, R), jnp.float32),
        jax.ShapeDtypeStruct((NP, R), jnp.int32),
    ),
    scratch_types=[
        pltpu.VMEM((NODES_W,), jnp.int32),
        pltpu.VMEM((NODES_W, R), jnp.int32),
        pltpu.VMEM((40, 128), jnp.int32),
        pltpu.VMEM((128, R), jnp.float32),
        pltpu.VMEM((8, R), jnp.float32),
        pltpu.SemaphoreType.DMA,
    ],
)
def _ctx_gather(cand_hbm, kglp_hbm, proj_hbm, xsum_hbm, tokout_hbm,
                cand_v, tok_v, idx_v, rows_v, acc_v, sem):
    base = _wid() * NODES_W
    pltpu.sync_copy(cand_hbm.at[pl.ds(base, NODES_W)], cand_v)
    hs = [
        pltpu.async_copy(kglp_hbm.at[cand_v.at[pl.ds(k * 40, 40)]],
                         tok_v.at[pl.ds(k * 40, 40)], sem)
        for k in range(8)
    ]
    for h in hs:
        h.wait()
    pltpu.sync_copy(tok_v, tokout_hbm.at[pl.ds(base, NODES_W)])

    for n in range(NODES_W):
        idx_v[n // 8, pl.ds((n % 8) * 16, 16)] = tok_v[n, pl.ds(0, 16)]

    def chunk(cch, _):
        hs2 = [
            pltpu.async_copy(proj_hbm.at[idx_v.at[cch, pl.ds(k * 16, 16)]],
                             rows_v.at[pl.ds(k * 16, 16)], sem)
            for k in range(8)
        ]
        for h in hs2:
            h.wait()
        for j in range(8):
            for v in range(R // 16):
                sl = pl.ds(v * 16, 16)
                s = rows_v[j * 16, sl]
                for l in range(1, 16):
                    s = s + rows_v[j * 16 + l, sl]
                acc_v[j, sl] = s
        pltpu.sync_copy(acc_v, xsum_hbm.at[pl.ds(base + cch * 8, 8)])
        return 0

    lax.fori_loop(0, 40, chunk, 0)


def _xfin_body(xs_ref, tok_ref, db_ref, o_ref):
    cnt = jnp.sum((tok_ref[...] != 0).astype(jnp.float32), axis=1, keepdims=True)
    x = xs_ref[...] / jnp.maximum(cnt, 1.0)
    o_ref[...] = x + jnp.where(cnt > 0.0, db_ref[...], 0.0)


def _x_finalize(xsum, tok, down_b):
    tile = 1024
    return pl.pallas_call(
        _xfin_body,
        grid=(NP // tile,),
        in_specs=[
            pl.BlockSpec((tile, R), lambda i: (i, 0)),
            pl.BlockSpec((tile, R), lambda i: (i, 0)),
            pl.BlockSpec((1, R), lambda i: (0, 0)),
        ],
        out_specs=pl.BlockSpec((tile, R), lambda i: (i, 0)),
        out_shape=jax.ShapeDtypeStruct((NP, R), jnp.float32),
    )(xsum, tok, down_b)


def _gate_body(x_ref, lh_ref, qw_ref, gwx_ref, gwq_ref, gb_ref, w1_ref, b1_ref,
               w2r_ref, b2_ref, pwt_ref, pwb_ref, xg_ref, a_ref, b_ref):
    q = jnp.dot(lh_ref[...], qw_ref[...], preferred_element_type=jnp.float32)
    x = x_ref[...]
    gi = jnp.dot(x, gwx_ref[...], preferred_element_type=jnp.float32)
    gi = gi + jnp.dot(q, gwq_ref[...], preferred_element_type=jnp.float32)
    gi = jnp.maximum(gi + gb_ref[...], 0.0)
    hm = jnp.maximum(
        jnp.dot(gi, w1_ref[...], preferred_element_type=jnp.float32) + b1_ref[...], 0.0)
    gl = jnp.sum(hm * w2r_ref[...], axis=1, keepdims=True) + b2_ref[0, 0]
    gate = 1.0 / (1.0 + jnp.exp(-gl))
    xg = x * gate
    xg_ref[...] = xg
    a_ref[...] = jnp.dot(xg, pwt_ref[...], preferred_element_type=jnp.float32)
    b_ref[...] = jnp.dot(xg, pwb_ref[...], preferred_element_type=jnp.float32)


def _gate_stage(x, lh, qw, gwx, gwq, gb, w1, b1, w2r, b2, pwt, pwb):
    tile = 1024
    full = lambda r, c: pl.BlockSpec((r, c), lambda i: (0, 0))
    nod = pl.BlockSpec((tile, R), lambda i: (i, 0))
    return pl.pallas_call(
        _gate_body,
        grid=(NP // tile,),
        in_specs=[nod, full(1, H), full(H, R), full(R, R), full(R, R),
                  full(1, R), full(R, R), full(1, R), full(1, R), full(1, R),
                  full(R, R), full(R, R)],
        out_specs=[nod, nod, nod],
        out_shape=[jax.ShapeDtypeStruct((NP, R), jnp.float32)] * 3,
    )(x, lh, qw, gwx, gwq, gb, w1, b1, w2r, b2, pwt, pwb)


@functools.partial(
    pl.kernel,
    mesh=_MESH,
    compiler_params=_SC_PARAMS,
    out_type=(
        jax.ShapeDtypeStruct((NTASK, NCH, SCAN), jnp.int32),
        jax.ShapeDtypeStruct((NTASK, NCH_PAD), jnp.int32),
    ),
    scratch_types=[
        pltpu.VMEM((SCAN,), jnp.int32),
        pltpu.VMEM((SCAN,), jnp.int32),
        pltpu.VMEM((SCAN + 16,), jnp.int32),
        pltpu.VMEM((NCH_PAD,), jnp.int32),
    ],
)
def _bin_edges(src_hbm, dst_hbm, bins_hbm, counts_hbm, srcv, dstv, stagev, cntv):
    w = _wid()
    lane = lax.iota(jnp.int32, 16)
    for tt in range(2):
        t = w * 2 + tt
        lo = t * NPT

        for j in range(NCH_PAD // 16):
            cntv[pl.ds(j * 16, 16)] = jnp.zeros((16,), jnp.int32)

        def chunk_body(c, _):
            pltpu.sync_copy(src_hbm.at[c], srcv)
            pltpu.sync_copy(dst_hbm.at[c], dstv)
            cnt = jnp.int32(0)
            for i in range(SCAN // 16):
                sl = pl.ds(i * 16, 16)
                d = dstv[sl]
                s = srcv[sl]
                m = (d >= lo) & (d < lo + NPT)
                pk = (s << 8) | (d - lo)
                mi = m.astype(jnp.int32)
                cs = plsc.cumsum(mi)
                pos = cnt + cs - mi
                plsc.store_scatter(stagev, [pos], pk, mask=m)
                cnt = cnt + cs[15]
            pltpu.sync_copy(stagev.at[pl.ds(0, SCAN)], bins_hbm.at[t, c])
            plsc.store_scatter(cntv, [_splat(c)], _splat(cnt), mask=(lane == 0))
            return 0

        lax.fori_loop(0, NCH, chunk_body, 0)
        pltpu.sync_copy(cntv, counts_hbm.at[t])


def _make_seg(compute_deg):
    outs = [
        jax.ShapeDtypeStruct((NP, R), jnp.float32),
        jax.ShapeDtypeStruct((NP, R), jnp.float32),
        jax.ShapeDtypeStruct((NP, R), jnp.float32),
        jax.ShapeDtypeStruct((NP, R), jnp.float32),
    ]
    scr = [
        pltpu.VMEM((NPT, R), jnp.float32),
        pltpu.VMEM((NPT, R), jnp.float32),
        pltpu.VMEM((NPT, R), jnp.float32),
        pltpu.VMEM((NPT, R), jnp.float32),
        pltpu.VMEM((GC, R), jnp.float32),
        pltpu.VMEM((GC,), jnp.int32),
        pltpu.VMEM((GC,), jnp.int32),
        pltpu.VMEM((NCH_PAD,), jnp.int32),
        pltpu.SemaphoreType.DMA,
    ]
    if compute_deg:
        outs.append(jax.ShapeDtypeStruct((NP, 16), jnp.float32))
        scr.insert(-1, pltpu.VMEM((NPT, 16), jnp.float32))

    @functools.partial(pl.kernel, mesh=_MESH, out_type=tuple(outs),
                       compiler_params=_SC_PARAMS, scratch_types=scr)
    def _seg(bins_hbm, counts_hbm, b_hbm, *refs):
        if compute_deg:
            (ss_hbm, sq_hbm, mn_hbm, mx_hbm, deg_hbm,
             asum, asq, amn, amx, rowsv, pkv, idxv, cntrow, degv, sem) = refs
        else:
            (ss_hbm, sq_hbm, mn_hbm, mx_hbm,
             asum, asq, amn, amx, rowsv, pkv, idxv, cntrow, sem) = refs
        w = _wid()
        lane = lax.iota(jnp.int32, 16)
        zero = jnp.zeros((16,), jnp.float32)
        lanec = [lane + v * 16 for v in range(R // 16)]
        for tt in range(2):
            t = w * 2 + tt
            lo = t * NPT

            def zb(j, _):
                jv = _splat(j)
                for v in range(R // 16):
                    plsc.store_scatter(asum, [jv, lanec[v]], zero)
                    plsc.store_scatter(asq, [jv, lanec[v]], zero)
                    plsc.store_scatter(amn, [jv, lanec[v]], zero + BIG)
                    plsc.store_scatter(amx, [jv, lanec[v]], zero - BIG)
                if compute_deg:
                    plsc.store_scatter(degv, [jv, lane], zero)
                return 0

            lax.fori_loop(0, NPT, zb, 0)
            pltpu.sync_copy(counts_hbm.at[t], cntrow)

            def cbody(c, _):
                cnt_c = _vextract(cntrow, c)

                def sbody(sub, _):
                    pltpu.sync_copy(bins_hbm.at[t, c, pl.ds(sub * GC, GC)], pkv)
                    base_s = _splat(sub * GC)
                    cnt_s = _splat(cnt_c)
                    for i in range(GC // 16):
                        sl = pl.ds(i * 16, 16)
                        pk = pkv[sl]
                        pos = base_s + (lane + i * 16)
                        idxv[sl] = jnp.where(pos < cnt_s, pk >> 8, 0)
                    hs = [
                        pltpu.async_copy(
                            b_hbm.at[idxv.at[pl.ds(k * 16, 16)]],
                            rowsv.at[pl.ds(k * 16, 16)], sem)
                        for k in range(GC // 16)
                    ]
                    for h in hs:
                        h.wait()
                    m = jnp.minimum(cnt_c - sub * GC, GC)

                    def ebody(j, _):
                        jv = _splat(j)
                        dv = plsc.load_gather(pkv, [jv]) & 255
                        for v in range(R // 16):
                            lc = lanec[v]
                            r = plsc.load_gather(rowsv, [jv, lc])
                            s0 = plsc.load_gather(asum, [dv, lc])
                            plsc.store_scatter(asum, [dv, lc], s0 + r)
                            q0 = plsc.load_gather(asq, [dv, lc])
                            plsc.store_scatter(asq, [dv, lc], q0 + r * r)
                            m0 = plsc.load_gather(amn, [dv, lc])
                            plsc.store_scatter(amn, [dv, lc], jnp.minimum(m0, r))
                            x0 = plsc.load_gather(amx, [dv, lc])
                            plsc.store_scatter(amx, [dv, lc], jnp.maximum(x0, r))
                        if compute_deg:
                            d0 = plsc.load_gather(degv, [dv, lane])
                            plsc.store_scatter(degv, [dv, lane], d0 + 1.0)
                        return 0

                    lax.fori_loop(0, m, ebody, 0)
                    return 0

                nsub = lax.shift_right_logical(cnt_c + (GC - 1), 7)
                lax.fori_loop(0, nsub, sbody, 0)
                return 0

            lax.fori_loop(0, NCH, cbody, 0)
            sl = pl.ds(lo, NPT)
            pltpu.sync_copy(asum, ss_hbm.at[sl])
            pltpu.sync_copy(asq, sq_hbm.at[sl])
            pltpu.sync_copy(amn, mn_hbm.at[sl])
            pltpu.sync_copy(amx, mx_hbm.at[sl])
            if compute_deg:
                pltpu.sync_copy(degv, deg_hbm.at[sl])

    return _seg


_seg_deg = _make_seg(True)
_seg_nodeg = _make_seg(False)


def _post_body(x_ref, xg_ref, a_ref, ss_ref, sq_ref, mn_ref, mx_ref, deg_ref,
               hist_ref, preb_ref, pw_ref, pb_ref, lw_ref, lb_ref, o_ref):
    hist = hist_ref[...]
    binsv = lax.broadcasted_iota(jnp.int32, (1, R), 1).astype(jnp.float32)
    avg_log = jnp.sum(jnp.log(binsv + 1.0) * hist) / jnp.sum(hist)
    deg = deg_ref[...][:, 0:1]
    degc = jnp.maximum(deg, 1.0)
    hase = deg > 0.0
    ab = a_ref[...] + preb_ref[...]
    ssum = ss_ref[...]
    mean = jnp.where(hase, ab, 0.0) + ssum / degc
    s2 = (deg * ab * ab + 2.0 * ab * ssum + sq_ref[...]) / degc
    std = jnp.sqrt(jnp.maximum(s2 - mean * mean, 0.0) + 1e-5)
    mn = jnp.where(hase, ab + mn_ref[...], 0.0)
    mx = jnp.where(hase, ab + mx_ref[...], 0.0)
    log_deg = jnp.log(degc + 1.0)
    s = log_deg / avg_log
    tt = avg_log / log_deg
    blocks = [xg_ref[...], mean, mn, mx, std, mean * s, mn * s, mx * s,
              std * s, mean * tt, mn * tt, mx * tt, std * tt]
    pw = pw_ref[...]
    out = pb_ref[...]
    for k in range(13):
        out = out + jnp.dot(blocks[k], pw[k * R:(k + 1) * R, :],
                            preferred_element_type=jnp.float32)
    out = jnp.dot(out, lw_ref[...], preferred_element_type=jnp.float32) + lb_ref[...]
    o_ref[...] = jnp.maximum(out + x_ref[...], 0.0)


def _post_stage(x, xg, a, ss, sq, mn, mx, deg, hist, preb, pw, pb, lw, lb):
    tile = 1024
    nod = pl.BlockSpec((tile, R), lambda i: (i, 0))
    full = lambda r, c: pl.BlockSpec((r, c), lambda i: (0, 0))
    return pl.pallas_call(
        _post_body,
        grid=(NP // tile,),
        in_specs=[nod, nod, nod, nod, nod, nod, nod,
                  pl.BlockSpec((tile, 16), lambda i: (i, 0)),
                  full(1, R), full(1, R), full(13 * R, R), full(1, R),
                  full(R, R), full(1, R)],
        out_specs=nod,
        out_shape=jax.ShapeDtypeStruct((NP, R), jnp.float32),
    )(x, xg, a, ss, sq, mn, mx, deg, hist, preb, pw, pb, lw, lb)


def _score_body(x_ref, sw_ref, sb_ref, o_ref):
    o_ref[...] = jnp.sum(x_ref[...] * sw_ref[...], axis=1) + sb_ref[0, 0]


def _score_stage(x, swr, sb):
    tile = 1024
    return pl.pallas_call(
        _score_body,
        grid=(NP // tile,),
        in_specs=[pl.BlockSpec((tile, R), lambda i: (i, 0)),
                  pl.BlockSpec((1, R), lambda i: (0, 0)),
                  pl.BlockSpec((1, R), lambda i: (0, 0))],
        out_specs=pl.BlockSpec((tile,), lambda i: (i,)),
        out_shape=jax.ShapeDtypeStruct((NP,), jnp.float32),
    )(x, swr, sb)


def kernel(llm_hidden_state, params, candidate_ids, edge_index, kgl2token_ids,
           deg_histogram):
    p = params
    f32 = jnp.float32

    proj = _proj_table(p["llm_emb"], p["down_w"])

    cand = jnp.pad(candidate_ids, (0, NP - N))
    kglp = jnp.pad(kgl2token_ids, ((0, 0), (0, R - L)))
    xsum, tokrows = _ctx_gather(cand, kglp, proj)
    x = _x_finalize(xsum, tokrows, p["down_b"].reshape(1, R))

    src = jnp.pad(edge_index[0], (0, EP - E)).reshape(NCH, SCAN)
    dst = jnp.pad(edge_index[1], (0, EP - E),
                  constant_values=1 << 29).reshape(NCH, SCAN)
    bins, counts = _bin_edges(src, dst)

    hist = jnp.zeros((1, R), f32).at[0, :deg_histogram.shape[0]].set(
        deg_histogram.astype(f32))
    lh = llm_hidden_state
    qw = p["query_w"]

    deg = None
    for i in range(2):
        gw = p[f"l{i}_gate_w"]
        xg, a, b = _gate_stage(
            x, lh, qw, gw[:R], gw[R:], p[f"l{i}_gate_b"].reshape(1, R),
            p[f"l{i}_gmlp_w1"], p[f"l{i}_gmlp_b1"].reshape(1, R),
            p[f"l{i}_gmlp_w2"].reshape(1, R),
            jnp.broadcast_to(p[f"l{i}_gmlp_b2"].reshape(1, 1), (1, R)),
            p[f"l{i}_pre_w"][:R], p[f"l{i}_pre_w"][R:])
        if i == 0:
            ss, sq, mn, mx, deg = _seg_deg(bins, counts, b)
        else:
            ss, sq, mn, mx = _seg_nodeg(bins, counts, b)
        x = _post_stage(
            x, xg, a, ss, sq, mn, mx, deg, hist,
            p[f"l{i}_pre_b"].reshape(1, R), p[f"l{i}_post_w"],
            p[f"l{i}_post_b"].reshape(1, R), p[f"l{i}_lin_w"],
            p[f"l{i}_lin_b"].reshape(1, R))

    logits = _score_stage(x, p["score_w"].reshape(1, R),
                          jnp.broadcast_to(p["score_b"].reshape(1, 1), (1, R)))
    return logits[:N]


# speedup vs baseline: 7.3022x; 7.3009x over previous
import functools

import jax
import jax.numpy as jnp
from jax import lax
from jax.experimental import pallas as pl
from jax.experimental.pallas import tpu as pltpu
from jax.experimental.pallas import tpu_sc as plsc

N = 10000
NP = 10240
E = 320000
EP = 321536
R = 128
H = 768
L = 8
VOCAB = 32000

_INFO = plsc.get_sparse_core_info()
NC = _INFO.num_cores
NS = _INFO.num_subcores
NWORK = NC * NS
NODES_W = NP // NWORK
NTASK = 64
NPT = NP // NTASK
SCAN = 2048
NCH = EP // SCAN
NCH_PAD = 160
GC = 128
BIG = 3.0e38

_MESH = plsc.VectorSubcoreMesh(core_axis_name="c", subcore_axis_name="s")
_SC_PARAMS = pltpu.CompilerParams(needs_layout_passes=False)


def _wid():
    return lax.axis_index("s") * NC + lax.axis_index("c")


def _vextract(ref, j):
    base = lax.shift_left(lax.shift_right_logical(j, 4), 4)
    vec = ref[pl.ds(base, 16)]
    lane = lax.iota(jnp.int32, 16)
    return jnp.sum(jnp.where(lane == (j - base), vec, 0))


def _splat(x):
    return jnp.zeros((16,), jnp.int32) + x


def _proj_body(emb_ref, w_ref, o_ref):
    i = pl.program_id(0)
    acc = jnp.dot(emb_ref[...], w_ref[...], preferred_element_type=jnp.float32)
    rows = lax.broadcasted_iota(jnp.int32, acc.shape, 0)
    o_ref[...] = jnp.where((rows == 0) & (i == 0), 0.0, acc)


def _proj_table(emb, w):
    tile = 2000
    return pl.pallas_call(
        _proj_body,
        grid=(VOCAB // tile,),
        in_specs=[
            pl.BlockSpec((tile, H), lambda i: (i, 0)),
            pl.BlockSpec((H, R), lambda i: (0, 0)),
        ],
        out_specs=pl.BlockSpec((tile, R), lambda i: (i, 0)),
        out_shape=jax.ShapeDtypeStruct((VOCAB, R), jnp.float32),
    )(emb, w)


@functools.partial(
    pl.kernel,
    mesh=_MESH,
    compiler_params=_SC_PARAMS,
    out_type=(
        jax.ShapeDtypeStruct((NP, R), jnp.float32),
        jax.ShapeDtypeStruct((NP, R), jnp.int32),
    ),
    scratch_types=[
        pltpu.VMEM((NODES_W,), jnp.int32),
        pltpu.VMEM((NODES_W, R), jnp.int32),
        pltpu.VMEM((40, 128), jnp.int32),
        pltpu.VMEM((128, R), jnp.float32),
        pltpu.VMEM((8, R), jnp.float32),
        pltpu.SemaphoreType.DMA,
    ],
)
def _ctx_gather(cand_hbm, kglp_hbm, proj_hbm, xsum_hbm, tokout_hbm,
                cand_v, tok_v, idx_v, rows_v, acc_v, sem):
    base = _wid() * NODES_W
    pltpu.sync_copy(cand_hbm.at[pl.ds(base, NODES_W)], cand_v)
    hs = [
        pltpu.async_copy(kglp_hbm.at[cand_v.at[pl.ds(k * 40, 40)]],
                         tok_v.at[pl.ds(k * 40, 40)], sem)
        for k in range(8)
    ]
    for h in hs:
        h.wait()
    pltpu.sync_copy(tok_v, tokout_hbm.at[pl.ds(base, NODES_W)])

    for n in range(NODES_W):
        idx_v[n // 8, pl.ds((n % 8) * 16, 16)] = tok_v[n, pl.ds(0, 16)]

    def chunk(cch, _):
        hs2 = [
            pltpu.async_copy(proj_hbm.at[idx_v.at[cch, pl.ds(k * 16, 16)]],
                             rows_v.at[pl.ds(k * 16, 16)], sem)
            for k in range(8)
        ]
        for h in hs2:
            h.wait()
        for j in range(8):
            for v in range(R // 16):
                sl = pl.ds(v * 16, 16)
                s = rows_v[j * 16, sl]
                for l in range(1, 16):
                    s = s + rows_v[j * 16 + l, sl]
                acc_v[j, sl] = s
        pltpu.sync_copy(acc_v, xsum_hbm.at[pl.ds(base + cch * 8, 8)])
        return 0

    lax.fori_loop(0, 40, chunk, 0)


def _xfin_body(xs_ref, tok_ref, db_ref, o_ref):
    cnt = jnp.sum((tok_ref[...] != 0).astype(jnp.float32), axis=1, keepdims=True)
    x = xs_ref[...] / jnp.maximum(cnt, 1.0)
    o_ref[...] = x + jnp.where(cnt > 0.0, db_ref[...], 0.0)


def _x_finalize(xsum, tok, down_b):
    tile = 1024
    return pl.pallas_call(
        _xfin_body,
        grid=(NP // tile,),
        in_specs=[
            pl.BlockSpec((tile, R), lambda i: (i, 0)),
            pl.BlockSpec((tile, R), lambda i: (i, 0)),
            pl.BlockSpec((1, R), lambda i: (0, 0)),
        ],
        out_specs=pl.BlockSpec((tile, R), lambda i: (i, 0)),
        out_shape=jax.ShapeDtypeStruct((NP, R), jnp.float32),
    )(xsum, tok, down_b)


def _gate_body(x_ref, lh_ref, qw_ref, gwx_ref, gwq_ref, gb_ref, w1_ref, b1_ref,
               w2r_ref, b2_ref, pwt_ref, pwb_ref, xg_ref, a_ref, b_ref):
    q = jnp.dot(lh_ref[...], qw_ref[...], preferred_element_type=jnp.float32)
    x = x_ref[...]
    gi = jnp.dot(x, gwx_ref[...], preferred_element_type=jnp.float32)
    gi = gi + jnp.dot(q, gwq_ref[...], preferred_element_type=jnp.float32)
    gi = jnp.maximum(gi + gb_ref[...], 0.0)
    hm = jnp.maximum(
        jnp.dot(gi, w1_ref[...], preferred_element_type=jnp.float32) + b1_ref[...], 0.0)
    gl = jnp.sum(hm * w2r_ref[...], axis=1, keepdims=True) + b2_ref[0, 0]
    gate = 1.0 / (1.0 + jnp.exp(-gl))
    xg = x * gate
    xg_ref[...] = xg
    a_ref[...] = jnp.dot(xg, pwt_ref[...], preferred_element_type=jnp.float32)
    b_ref[...] = jnp.dot(xg, pwb_ref[...], preferred_element_type=jnp.float32)


def _gate_stage(x, lh, qw, gwx, gwq, gb, w1, b1, w2r, b2, pwt, pwb):
    tile = 1024
    full = lambda r, c: pl.BlockSpec((r, c), lambda i: (0, 0))
    nod = pl.BlockSpec((tile, R), lambda i: (i, 0))
    return pl.pallas_call(
        _gate_body,
        grid=(NP // tile,),
        in_specs=[nod, full(1, H), full(H, R), full(R, R), full(R, R),
                  full(1, R), full(R, R), full(1, R), full(1, R), full(1, R),
                  full(R, R), full(R, R)],
        out_specs=[nod, nod, nod],
        out_shape=[jax.ShapeDtypeStruct((NP, R), jnp.float32)] * 3,
    )(x, lh, qw, gwx, gwq, gb, w1, b1, w2r, b2, pwt, pwb)


@functools.partial(
    pl.kernel,
    mesh=_MESH,
    compiler_params=_SC_PARAMS,
    out_type=(
        jax.ShapeDtypeStruct((NTASK, NCH, SCAN), jnp.int32),
        jax.ShapeDtypeStruct((NTASK, NCH_PAD), jnp.int32),
    ),
    scratch_types=[
        pltpu.VMEM((SCAN,), jnp.int32),
        pltpu.VMEM((SCAN,), jnp.int32),
        pltpu.VMEM((SCAN + 16,), jnp.int32),
        pltpu.VMEM((NCH_PAD,), jnp.int32),
    ],
)
def _bin_edges(src_hbm, dst_hbm, bins_hbm, counts_hbm, srcv, dstv, stagev, cntv):
    w = _wid()
    lane = lax.iota(jnp.int32, 16)
    for tt in range(2):
        t = w * 2 + tt
        lo = t * NPT

        for j in range(NCH_PAD // 16):
            cntv[pl.ds(j * 16, 16)] = jnp.zeros((16,), jnp.int32)

        def chunk_body(c, _):
            pltpu.sync_copy(src_hbm.at[c], srcv)
            pltpu.sync_copy(dst_hbm.at[c], dstv)
            cnt = jnp.int32(0)
            for i in range(SCAN // 16):
                sl = pl.ds(i * 16, 16)
                d = dstv[sl]
                s = srcv[sl]
                m = (d >= lo) & (d < lo + NPT)
                pk = (s << 8) | (d - lo)
                mi = m.astype(jnp.int32)
                cs = plsc.cumsum(mi)
                pos = cnt + cs - mi
                plsc.store_scatter(stagev, [pos], pk, mask=m)
                cnt = cnt + cs[15]
            pltpu.sync_copy(stagev.at[pl.ds(0, SCAN)], bins_hbm.at[t, c])
            plsc.store_scatter(cntv, [_splat(c)], _splat(cnt), mask=(lane == 0))
            return 0

        lax.fori_loop(0, NCH, chunk_body, 0)
        pltpu.sync_copy(cntv, counts_hbm.at[t])


def _make_seg(compute_deg):
    outs = [
        jax.ShapeDtypeStruct((NP, R), jnp.float32),
        jax.ShapeDtypeStruct((NP, R), jnp.float32),
        jax.ShapeDtypeStruct((NP, R), jnp.float32),
        jax.ShapeDtypeStruct((NP, R), jnp.float32),
    ]
    scr = [
        pltpu.VMEM((NPT, R), jnp.float32),
        pltpu.VMEM((NPT, R), jnp.float32),
        pltpu.VMEM((NPT, R), jnp.float32),
        pltpu.VMEM((NPT, R), jnp.float32),
        pltpu.VMEM((GC, R), jnp.float32),
        pltpu.VMEM((GC,), jnp.int32),
        pltpu.VMEM((GC,), jnp.int32),
        pltpu.VMEM((NCH_PAD,), jnp.int32),
        pltpu.SemaphoreType.DMA,
    ]
    if compute_deg:
        outs.append(jax.ShapeDtypeStruct((NP, 16), jnp.float32))
        scr.insert(-1, pltpu.VMEM((NPT, 16), jnp.float32))

    @functools.partial(pl.kernel, mesh=_MESH, out_type=tuple(outs),
                       compiler_params=_SC_PARAMS, scratch_types=scr)
    def _seg(bins_hbm, counts_hbm, b_hbm, *refs):
        if compute_deg:
            (ss_hbm, sq_hbm, mn_hbm, mx_hbm, deg_hbm,
             asum, asq, amn, amx, rowsv, pkv, idxv, cntrow, degv, sem) = refs
        else:
            (ss_hbm, sq_hbm, mn_hbm, mx_hbm,
             asum, asq, amn, amx, rowsv, pkv, idxv, cntrow, sem) = refs
        w = _wid()
        lane = lax.iota(jnp.int32, 16)
        zero = jnp.zeros((16,), jnp.float32)
        lanec = [lane + v * 16 for v in range(R // 16)]
        for tt in range(2):
            t = w * 2 + tt
            lo = t * NPT

            def zb(j, _):
                jv = _splat(j)
                for v in range(R // 16):
                    plsc.store_scatter(asum, [jv, lanec[v]], zero)
                    plsc.store_scatter(asq, [jv, lanec[v]], zero)
                    plsc.store_scatter(amn, [jv, lanec[v]], zero + BIG)
                    plsc.store_scatter(amx, [jv, lanec[v]], zero - BIG)
                if compute_deg:
                    plsc.store_scatter(degv, [jv, lane], zero)
                return 0

            lax.fori_loop(0, NPT, zb, 0)
            pltpu.sync_copy(counts_hbm.at[t], cntrow)

            def cbody(c, _):
                cnt_c = _vextract(cntrow, c)

                def sbody(sub, _):
                    pltpu.sync_copy(bins_hbm.at[t, c, pl.ds(sub * GC, GC)], pkv)
                    base_s = _splat(sub * GC)
                    cnt_s = _splat(cnt_c)
                    for i in range(GC // 16):
                        sl = pl.ds(i * 16, 16)
                        pk = pkv[sl]
                        pos = base_s + (lane + i * 16)
                        idxv[sl] = jnp.where(pos < cnt_s, pk >> 8, 0)
                    need = cnt_c - sub * GC
                    for k in range(GC // 16):
                        @pl.when(need > k * 16)
                        def _fire(k=k):
                            pltpu.async_copy(
                                b_hbm.at[idxv.at[pl.ds(k * 16, 16)]],
                                rowsv.at[pl.ds(k * 16, 16)], sem).wait()
                    m = jnp.minimum(need, GC)

                    def ebody(j, _):
                        jv = _splat(j)
                        dv = plsc.load_gather(pkv, [jv]) & 255
                        for v in range(R // 16):
                            lc = lanec[v]
                            r = plsc.load_gather(rowsv, [jv, lc])
                            s0 = plsc.load_gather(asum, [dv, lc])
                            plsc.store_scatter(asum, [dv, lc], s0 + r)
                            q0 = plsc.load_gather(asq, [dv, lc])
                            plsc.store_scatter(asq, [dv, lc], q0 + r * r)
                            m0 = plsc.load_gather(amn, [dv, lc])
                            plsc.store_scatter(amn, [dv, lc], jnp.minimum(m0, r))
                            x0 = plsc.load_gather(amx, [dv, lc])
                            plsc.store_scatter(amx, [dv, lc], jnp.maximum(x0, r))
                        if compute_deg:
                            d0 = plsc.load_gather(degv, [dv, lane])
                            plsc.store_scatter(degv, [dv, lane], d0 + 1.0)
                        return 0

                    lax.fori_loop(0, m, ebody, 0)
                    return 0

                nsub = lax.shift_right_logical(cnt_c + (GC - 1), 7)
                lax.fori_loop(0, nsub, sbody, 0)
                return 0

            lax.fori_loop(0, NCH, cbody, 0)
            sl = pl.ds(lo, NPT)
            pltpu.sync_copy(asum, ss_hbm.at[sl])
            pltpu.sync_copy(asq, sq_hbm.at[sl])
            pltpu.sync_copy(amn, mn_hbm.at[sl])
            pltpu.sync_copy(amx, mx_hbm.at[sl])
            if compute_deg:
                pltpu.sync_copy(degv, deg_hbm.at[sl])

    return _seg


_seg_deg = _make_seg(True)
_seg_nodeg = _make_seg(False)


def _post_body(x_ref, xg_ref, a_ref, ss_ref, sq_ref, mn_ref, mx_ref, deg_ref,
               hist_ref, preb_ref, pw_ref, pb_ref, lw_ref, lb_ref, o_ref):
    hist = hist_ref[...]
    binsv = lax.broadcasted_iota(jnp.int32, (1, R), 1).astype(jnp.float32)
    avg_log = jnp.sum(jnp.log(binsv + 1.0) * hist) / jnp.sum(hist)
    deg = deg_ref[...][:, 0:1]
    degc = jnp.maximum(deg, 1.0)
    hase = deg > 0.0
    ab = a_ref[...] + preb_ref[...]
    ssum = ss_ref[...]
    mean = jnp.where(hase, ab, 0.0) + ssum / degc
    s2 = (deg * ab * ab + 2.0 * ab * ssum + sq_ref[...]) / degc
    std = jnp.sqrt(jnp.maximum(s2 - mean * mean, 0.0) + 1e-5)
    mn = jnp.where(hase, ab + mn_ref[...], 0.0)
    mx = jnp.where(hase, ab + mx_ref[...], 0.0)
    log_deg = jnp.log(degc + 1.0)
    s = log_deg / avg_log
    tt = avg_log / log_deg
    blocks = [xg_ref[...], mean, mn, mx, std, mean * s, mn * s, mx * s,
              std * s, mean * tt, mn * tt, mx * tt, std * tt]
    pw = pw_ref[...]
    out = pb_ref[...]
    for k in range(13):
        out = out + jnp.dot(blocks[k], pw[k * R:(k + 1) * R, :],
                            preferred_element_type=jnp.float32)
    out = jnp.dot(out, lw_ref[...], preferred_element_type=jnp.float32) + lb_ref[...]
    o_ref[...] = jnp.maximum(out + x_ref[...], 0.0)


def _post_stage(x, xg, a, ss, sq, mn, mx, deg, hist, preb, pw, pb, lw, lb):
    tile = 1024
    nod = pl.BlockSpec((tile, R), lambda i: (i, 0))
    full = lambda r, c: pl.BlockSpec((r, c), lambda i: (0, 0))
    return pl.pallas_call(
        _post_body,
        grid=(NP // tile,),
        in_specs=[nod, nod, nod, nod, nod, nod, nod,
                  pl.BlockSpec((tile, 16), lambda i: (i, 0)),
                  full(1, R), full(1, R), full(13 * R, R), full(1, R),
                  full(R, R), full(1, R)],
        out_specs=nod,
        out_shape=jax.ShapeDtypeStruct((NP, R), jnp.float32),
    )(x, xg, a, ss, sq, mn, mx, deg, hist, preb, pw, pb, lw, lb)


def _score_body(x_ref, sw_ref, sb_ref, o_ref):
    o_ref[...] = jnp.sum(x_ref[...] * sw_ref[...], axis=1) + sb_ref[0, 0]


def _score_stage(x, swr, sb):
    tile = 1024
    return pl.pallas_call(
        _score_body,
        grid=(NP // tile,),
        in_specs=[pl.BlockSpec((tile, R), lambda i: (i, 0)),
                  pl.BlockSpec((1, R), lambda i: (0, 0)),
                  pl.BlockSpec((1, R), lambda i: (0, 0))],
        out_specs=pl.BlockSpec((tile,), lambda i: (i,)),
        out_shape=jax.ShapeDtypeStruct((NP,), jnp.float32),
    )(x, swr, sb)


def kernel(llm_hidden_state, params, candidate_ids, edge_index, kgl2token_ids,
           deg_histogram):
    p = params
    f32 = jnp.float32

    proj = _proj_table(p["llm_emb"], p["down_w"])

    cand = jnp.pad(candidate_ids, (0, NP - N))
    kglp = jnp.pad(kgl2token_ids, ((0, 0), (0, R - L)))
    xsum, tokrows = _ctx_gather(cand, kglp, proj)
    x = _x_finalize(xsum, tokrows, p["down_b"].reshape(1, R))

    src = jnp.pad(edge_index[0], (0, EP - E)).reshape(NCH, SCAN)
    dst = jnp.pad(edge_index[1], (0, EP - E),
                  constant_values=1 << 29).reshape(NCH, SCAN)
    bins, counts = _bin_edges(src, dst)

    hist = jnp.zeros((1, R), f32).at[0, :deg_histogram.shape[0]].set(
        deg_histogram.astype(f32))
    lh = llm_hidden_state
    qw = p["query_w"]

    deg = None
    for i in range(2):
        gw = p[f"l{i}_gate_w"]
        xg, a, b = _gate_stage(
            x, lh, qw, gw[:R], gw[R:], p[f"l{i}_gate_b"].reshape(1, R),
            p[f"l{i}_gmlp_w1"], p[f"l{i}_gmlp_b1"].reshape(1, R),
            p[f"l{i}_gmlp_w2"].reshape(1, R),
            jnp.broadcast_to(p[f"l{i}_gmlp_b2"].reshape(1, 1), (1, R)),
            p[f"l{i}_pre_w"][:R], p[f"l{i}_pre_w"][R:])
        if i == 0:
            ss, sq, mn, mx, deg = _seg_deg(bins, counts, b)
        else:
            ss, sq, mn, mx = _seg_nodeg(bins, counts, b)
        x = _post_stage(
            x, xg, a, ss, sq, mn, mx, deg, hist,
            p[f"l{i}_pre_b"].reshape(1, R), p[f"l{i}_post_w"],
            p[f"l{i}_post_b"].reshape(1, R), p[f"l{i}_lin_w"],
            p[f"l{i}_lin_b"].reshape(1, R))

    logits = _score_stage(x, p["score_w"].reshape(1, R),
                          jnp.broadcast_to(p["score_b"].reshape(1, 1), (1, R)))
    return logits[:N]


# speedup vs baseline: 10.3123x; 1.4122x over previous
import functools

import jax
import jax.numpy as jnp
from jax import lax
from jax.experimental import pallas as pl
from jax.experimental.pallas import tpu as pltpu
from jax.experimental.pallas import tpu_sc as plsc

N = 10000
NP = 10240
E = 320000
EP = 321536
R = 128
H = 768
L = 8
VOCAB = 32000

_INFO = plsc.get_sparse_core_info()
NC = _INFO.num_cores
NS = _INFO.num_subcores
NWORK = NC * NS
NODES_W = NP // NWORK
NTASK = 64
NPT = NP // NTASK
SCAN = 2048
NCH = EP // SCAN
NCH_PAD = 160
GC = 128
BIG = 3.0e38

_MESH = plsc.VectorSubcoreMesh(core_axis_name="c", subcore_axis_name="s")
_SC_PARAMS = pltpu.CompilerParams(needs_layout_passes=False)


def _wid():
    return lax.axis_index("s") * NC + lax.axis_index("c")


def _vextract(ref, j):
    base = lax.shift_left(lax.shift_right_logical(j, 4), 4)
    vec = ref[pl.ds(base, 16)]
    lane = lax.iota(jnp.int32, 16)
    return jnp.sum(jnp.where(lane == (j - base), vec, 0))


def _splat(x):
    return jnp.zeros((16,), jnp.int32) + x


def _proj_body(emb_ref, w_ref, o_ref):
    i = pl.program_id(0)
    acc = jnp.dot(emb_ref[...], w_ref[...], preferred_element_type=jnp.float32)
    rows = lax.broadcasted_iota(jnp.int32, acc.shape, 0)
    o_ref[...] = jnp.where((rows == 0) & (i == 0), 0.0, acc)


def _proj_table(emb, w):
    tile = 2000
    return pl.pallas_call(
        _proj_body,
        grid=(VOCAB // tile,),
        in_specs=[
            pl.BlockSpec((tile, H), lambda i: (i, 0)),
            pl.BlockSpec((H, R), lambda i: (0, 0)),
        ],
        out_specs=pl.BlockSpec((tile, R), lambda i: (i, 0)),
        out_shape=jax.ShapeDtypeStruct((VOCAB, R), jnp.float32),
    )(emb, w)


@functools.partial(
    pl.kernel,
    mesh=_MESH,
    compiler_params=_SC_PARAMS,
    out_type=(
        jax.ShapeDtypeStruct((NP, R), jnp.float32),
        jax.ShapeDtypeStruct((NP, R), jnp.int32),
    ),
    scratch_types=[
        pltpu.VMEM((NODES_W,), jnp.int32),
        pltpu.VMEM((NODES_W, R), jnp.int32),
        pltpu.VMEM((20, 128), jnp.int32),
        pltpu.VMEM((128, R), jnp.float32),
        pltpu.VMEM((16, R), jnp.float32),
        pltpu.SemaphoreType.DMA,
    ],
)
def _ctx_gather(cand_hbm, kglp_hbm, proj_hbm, xsum_hbm, tokout_hbm,
                cand_v, tok_v, idx_v, rows_v, acc_v, sem):
    base = _wid() * NODES_W
    pltpu.sync_copy(cand_hbm.at[pl.ds(base, NODES_W)], cand_v)
    hs = [
        pltpu.async_copy(kglp_hbm.at[cand_v.at[pl.ds(k * 40, 40)]],
                         tok_v.at[pl.ds(k * 40, 40)], sem)
        for k in range(8)
    ]
    for h in hs:
        h.wait()
    pltpu.sync_copy(tok_v, tokout_hbm.at[pl.ds(base, NODES_W)])

    lane = lax.iota(jnp.int32, 16)
    for p in range(NODES_W // 2):
        a = tok_v[2 * p, pl.ds(0, 16)]
        b = plsc.load_gather(tok_v, [_splat(2 * p + 1), lane & 7])
        packed = jnp.where(lane < 8, a, b)
        idx_v[p // 8, pl.ds((p % 8) * 16, 16)] = packed

    def chunk(cch, _):
        hs2 = [
            pltpu.async_copy(proj_hbm.at[idx_v.at[cch, pl.ds(k * 16, 16)]],
                             rows_v.at[pl.ds(k * 16, 16)], sem)
            for k in range(8)
        ]
        for h in hs2:
            h.wait()
        for j in range(16):
            for v in range(R // 16):
                sl = pl.ds(v * 16, 16)
                s = rows_v[j * 8, sl]
                for l in range(1, 8):
                    s = s + rows_v[j * 8 + l, sl]
                acc_v[j, sl] = s
        pltpu.sync_copy(acc_v, xsum_hbm.at[pl.ds(base + cch * 16, 16)])
        return 0

    lax.fori_loop(0, 20, chunk, 0)


def _xfin_body(xs_ref, tok_ref, db_ref, o_ref):
    cnt = jnp.sum((tok_ref[...] != 0).astype(jnp.float32), axis=1, keepdims=True)
    x = xs_ref[...] / jnp.maximum(cnt, 1.0)
    o_ref[...] = x + jnp.where(cnt > 0.0, db_ref[...], 0.0)


def _x_finalize(xsum, tok, down_b):
    tile = 1024
    return pl.pallas_call(
        _xfin_body,
        grid=(NP // tile,),
        in_specs=[
            pl.BlockSpec((tile, R), lambda i: (i, 0)),
            pl.BlockSpec((tile, R), lambda i: (i, 0)),
            pl.BlockSpec((1, R), lambda i: (0, 0)),
        ],
        out_specs=pl.BlockSpec((tile, R), lambda i: (i, 0)),
        out_shape=jax.ShapeDtypeStruct((NP, R), jnp.float32),
    )(xsum, tok, down_b)


def _gate_body(x_ref, lh_ref, qw_ref, gwx_ref, gwq_ref, gb_ref, w1_ref, b1_ref,
               w2r_ref, b2_ref, pwt_ref, pwb_ref, xg_ref, a_ref, b_ref):
    q = jnp.dot(lh_ref[...], qw_ref[...], preferred_element_type=jnp.float32)
    x = x_ref[...]
    gi = jnp.dot(x, gwx_ref[...], preferred_element_type=jnp.float32)
    gi = gi + jnp.dot(q, gwq_ref[...], preferred_element_type=jnp.float32)
    gi = jnp.maximum(gi + gb_ref[...], 0.0)
    hm = jnp.maximum(
        jnp.dot(gi, w1_ref[...], preferred_element_type=jnp.float32) + b1_ref[...], 0.0)
    gl = jnp.sum(hm * w2r_ref[...], axis=1, keepdims=True) + b2_ref[0, 0]
    gate = 1.0 / (1.0 + jnp.exp(-gl))
    xg = x * gate
    xg_ref[...] = xg
    a_ref[...] = jnp.dot(xg, pwt_ref[...], preferred_element_type=jnp.float32)
    b_ref[...] = jnp.dot(xg, pwb_ref[...], preferred_element_type=jnp.float32)


def _gate_stage(x, lh, qw, gwx, gwq, gb, w1, b1, w2r, b2, pwt, pwb):
    tile = 1024
    full = lambda r, c: pl.BlockSpec((r, c), lambda i: (0, 0))
    nod = pl.BlockSpec((tile, R), lambda i: (i, 0))
    return pl.pallas_call(
        _gate_body,
        grid=(NP // tile,),
        in_specs=[nod, full(1, H), full(H, R), full(R, R), full(R, R),
                  full(1, R), full(R, R), full(1, R), full(1, R), full(1, R),
                  full(R, R), full(R, R)],
        out_specs=[nod, nod, nod],
        out_shape=[jax.ShapeDtypeStruct((NP, R), jnp.float32)] * 3,
    )(x, lh, qw, gwx, gwq, gb, w1, b1, w2r, b2, pwt, pwb)


@functools.partial(
    pl.kernel,
    mesh=_MESH,
    compiler_params=_SC_PARAMS,
    out_type=(
        jax.ShapeDtypeStruct((NTASK, NCH, SCAN), jnp.int32),
        jax.ShapeDtypeStruct((NTASK, NCH_PAD), jnp.int32),
    ),
    scratch_types=[
        pltpu.VMEM((SCAN,), jnp.int32),
        pltpu.VMEM((SCAN,), jnp.int32),
        pltpu.VMEM((SCAN + 16,), jnp.int32),
        pltpu.VMEM((NCH_PAD,), jnp.int32),
    ],
)
def _bin_edges(src_hbm, dst_hbm, bins_hbm, counts_hbm, srcv, dstv, stagev, cntv):
    w = _wid()
    lane = lax.iota(jnp.int32, 16)
    for tt in range(2):
        t = w * 2 + tt
        lo = t * NPT

        for j in range(NCH_PAD // 16):
            cntv[pl.ds(j * 16, 16)] = jnp.zeros((16,), jnp.int32)

        def chunk_body(c, _):
            pltpu.sync_copy(src_hbm.at[c], srcv)
            pltpu.sync_copy(dst_hbm.at[c], dstv)
            cnt = jnp.int32(0)
            for i in range(SCAN // 16):
                sl = pl.ds(i * 16, 16)
                d = dstv[sl]
                s = srcv[sl]
                m = (d >= lo) & (d < lo + NPT)
                pk = (s << 8) | (d - lo)
                mi = m.astype(jnp.int32)
                cs = plsc.cumsum(mi)
                pos = cnt + cs - mi
                plsc.store_scatter(stagev, [pos], pk, mask=m)
                cnt = cnt + cs[15]
            pltpu.sync_copy(stagev.at[pl.ds(0, SCAN)], bins_hbm.at[t, c])
            plsc.store_scatter(cntv, [_splat(c)], _splat(cnt), mask=(lane == 0))
            return 0

        lax.fori_loop(0, NCH, chunk_body, 0)
        pltpu.sync_copy(cntv, counts_hbm.at[t])


def _make_seg(compute_deg):
    outs = [
        jax.ShapeDtypeStruct((NP, R), jnp.float32),
        jax.ShapeDtypeStruct((NP, R), jnp.float32),
        jax.ShapeDtypeStruct((NP, R), jnp.float32),
        jax.ShapeDtypeStruct((NP, R), jnp.float32),
    ]
    scr = [
        pltpu.VMEM((NPT, R), jnp.float32),
        pltpu.VMEM((NPT, R), jnp.float32),
        pltpu.VMEM((NPT, R), jnp.float32),
        pltpu.VMEM((NPT, R), jnp.float32),
        pltpu.VMEM((GC, R), jnp.float32),
        pltpu.VMEM((GC,), jnp.int32),
        pltpu.VMEM((GC,), jnp.int32),
        pltpu.VMEM((NCH_PAD,), jnp.int32),
        pltpu.SemaphoreType.DMA,
    ]
    if compute_deg:
        outs.append(jax.ShapeDtypeStruct((NP, 16), jnp.float32))
        scr.insert(-1, pltpu.VMEM((NPT, 16), jnp.float32))

    @functools.partial(pl.kernel, mesh=_MESH, out_type=tuple(outs),
                       compiler_params=_SC_PARAMS, scratch_types=scr)
    def _seg(bins_hbm, counts_hbm, b_hbm, *refs):
        if compute_deg:
            (ss_hbm, sq_hbm, mn_hbm, mx_hbm, deg_hbm,
             asum, asq, amn, amx, rowsv, pkv, idxv, cntrow, degv, sem) = refs
        else:
            (ss_hbm, sq_hbm, mn_hbm, mx_hbm,
             asum, asq, amn, amx, rowsv, pkv, idxv, cntrow, sem) = refs
        w = _wid()
        lane = lax.iota(jnp.int32, 16)
        zero = jnp.zeros((16,), jnp.float32)
        lanec = [lane + v * 16 for v in range(R // 16)]
        for tt in range(2):
            t = w * 2 + tt
            lo = t * NPT

            def zb(j, _):
                jv = _splat(j)
                for v in range(R // 16):
                    plsc.store_scatter(asum, [jv, lanec[v]], zero)
                    plsc.store_scatter(asq, [jv, lanec[v]], zero)
                    plsc.store_scatter(amn, [jv, lanec[v]], zero + BIG)
                    plsc.store_scatter(amx, [jv, lanec[v]], zero - BIG)
                if compute_deg:
                    plsc.store_scatter(degv, [jv, lane], zero)
                return 0

            lax.fori_loop(0, NPT, zb, 0)
            pltpu.sync_copy(counts_hbm.at[t], cntrow)

            def cbody(c, _):
                cnt_c = _vextract(cntrow, c)

                def sbody(sub, _):
                    pltpu.sync_copy(bins_hbm.at[t, c, pl.ds(sub * GC, GC)], pkv)
                    base_s = _splat(sub * GC)
                    cnt_s = _splat(cnt_c)
                    for i in range(GC // 16):
                        sl = pl.ds(i * 16, 16)
                        pk = pkv[sl]
                        pos = base_s + (lane + i * 16)
                        idxv[sl] = jnp.where(pos < cnt_s, pk >> 8, 0)
                    need = cnt_c - sub * GC
                    for k in range(GC // 16):
                        @pl.when(need > k * 16)
                        def _fire(k=k):
                            pltpu.async_copy(
                                b_hbm.at[idxv.at[pl.ds(k * 16, 16)]],
                                rowsv.at[pl.ds(k * 16, 16)], sem).wait()
                    m = jnp.minimum(need, GC)

                    def ebody(j, _):
                        jv = _splat(j)
                        dv = plsc.load_gather(pkv, [jv]) & 255
                        for v in range(R // 16):
                            lc = lanec[v]
                            r = plsc.load_gather(rowsv, [jv, lc])
                            s0 = plsc.load_gather(asum, [dv, lc])
                            plsc.store_scatter(asum, [dv, lc], s0 + r)
                            q0 = plsc.load_gather(asq, [dv, lc])
                            plsc.store_scatter(asq, [dv, lc], q0 + r * r)
                            m0 = plsc.load_gather(amn, [dv, lc])
                            plsc.store_scatter(amn, [dv, lc], jnp.minimum(m0, r))
                            x0 = plsc.load_gather(amx, [dv, lc])
                            plsc.store_scatter(amx, [dv, lc], jnp.maximum(x0, r))
                        if compute_deg:
                            d0 = plsc.load_gather(degv, [dv, lane])
                            plsc.store_scatter(degv, [dv, lane], d0 + 1.0)
                        return 0

                    lax.fori_loop(0, m, ebody, 0)
                    return 0

                nsub = lax.shift_right_logical(cnt_c + (GC - 1), 7)
                lax.fori_loop(0, nsub, sbody, 0)
                return 0

            lax.fori_loop(0, NCH, cbody, 0)
            sl = pl.ds(lo, NPT)
            pltpu.sync_copy(asum, ss_hbm.at[sl])
            pltpu.sync_copy(asq, sq_hbm.at[sl])
            pltpu.sync_copy(amn, mn_hbm.at[sl])
            pltpu.sync_copy(amx, mx_hbm.at[sl])
            if compute_deg:
                pltpu.sync_copy(degv, deg_hbm.at[sl])

    return _seg


_seg_deg = _make_seg(True)
_seg_nodeg = _make_seg(False)


def _post_body(x_ref, xg_ref, a_ref, ss_ref, sq_ref, mn_ref, mx_ref, deg_ref,
               hist_ref, preb_ref, pw_ref, pb_ref, lw_ref, lb_ref, o_ref):
    hist = hist_ref[...]
    binsv = lax.broadcasted_iota(jnp.int32, (1, R), 1).astype(jnp.float32)
    avg_log = jnp.sum(jnp.log(binsv + 1.0) * hist) / jnp.sum(hist)
    deg = deg_ref[...][:, 0:1]
    degc = jnp.maximum(deg, 1.0)
    hase = deg > 0.0
    ab = a_ref[...] + preb_ref[...]
    ssum = ss_ref[...]
    mean = jnp.where(hase, ab, 0.0) + ssum / degc
    s2 = (deg * ab * ab + 2.0 * ab * ssum + sq_ref[...]) / degc
    std = jnp.sqrt(jnp.maximum(s2 - mean * mean, 0.0) + 1e-5)
    mn = jnp.where(hase, ab + mn_ref[...], 0.0)
    mx = jnp.where(hase, ab + mx_ref[...], 0.0)
    log_deg = jnp.log(degc + 1.0)
    s = log_deg / avg_log
    tt = avg_log / log_deg
    blocks = [xg_ref[...], mean, mn, mx, std, mean * s, mn * s, mx * s,
              std * s, mean * tt, mn * tt, mx * tt, std * tt]
    pw = pw_ref[...]
    out = pb_ref[...]
    for k in range(13):
        out = out + jnp.dot(blocks[k], pw[k * R:(k + 1) * R, :],
                            preferred_element_type=jnp.float32)
    out = jnp.dot(out, lw_ref[...], preferred_element_type=jnp.float32) + lb_ref[...]
    o_ref[...] = jnp.maximum(out + x_ref[...], 0.0)


def _post_stage(x, xg, a, ss, sq, mn, mx, deg, hist, preb, pw, pb, lw, lb):
    tile = 1024
    nod = pl.BlockSpec((tile, R), lambda i: (i, 0))
    full = lambda r, c: pl.BlockSpec((r, c), lambda i: (0, 0))
    return pl.pallas_call(
        _post_body,
        grid=(NP // tile,),
        in_specs=[nod, nod, nod, nod, nod, nod, nod,
                  pl.BlockSpec((tile, 16), lambda i: (i, 0)),
                  full(1, R), full(1, R), full(13 * R, R), full(1, R),
                  full(R, R), full(1, R)],
        out_specs=nod,
        out_shape=jax.ShapeDtypeStruct((NP, R), jnp.float32),
    )(x, xg, a, ss, sq, mn, mx, deg, hist, preb, pw, pb, lw, lb)


def _score_body(x_ref, sw_ref, sb_ref, o_ref):
    o_ref[...] = jnp.sum(x_ref[...] * sw_ref[...], axis=1) + sb_ref[0, 0]


def _score_stage(x, swr, sb):
    tile = 1024
    return pl.pallas_call(
        _score_body,
        grid=(NP // tile,),
        in_specs=[pl.BlockSpec((tile, R), lambda i: (i, 0)),
                  pl.BlockSpec((1, R), lambda i: (0, 0)),
                  pl.BlockSpec((1, R), lambda i: (0, 0))],
        out_specs=pl.BlockSpec((tile,), lambda i: (i,)),
        out_shape=jax.ShapeDtypeStruct((NP,), jnp.float32),
    )(x, swr, sb)


def kernel(llm_hidden_state, params, candidate_ids, edge_index, kgl2token_ids,
           deg_histogram):
    p = params
    f32 = jnp.float32

    proj = _proj_table(p["llm_emb"], p["down_w"])

    cand = jnp.pad(candidate_ids, (0, NP - N))
    kglp = jnp.pad(kgl2token_ids, ((0, 0), (0, R - L)))
    xsum, tokrows = _ctx_gather(cand, kglp, proj)
    x = _x_finalize(xsum, tokrows, p["down_b"].reshape(1, R))

    src = jnp.pad(edge_index[0], (0, EP - E)).reshape(NCH, SCAN)
    dst = jnp.pad(edge_index[1], (0, EP - E),
                  constant_values=1 << 29).reshape(NCH, SCAN)
    bins, counts = _bin_edges(src, dst)

    hist = jnp.zeros((1, R), f32).at[0, :deg_histogram.shape[0]].set(
        deg_histogram.astype(f32))
    lh = llm_hidden_state
    qw = p["query_w"]

    deg = None
    for i in range(2):
        gw = p[f"l{i}_gate_w"]
        xg, a, b = _gate_stage(
            x, lh, qw, gw[:R], gw[R:], p[f"l{i}_gate_b"].reshape(1, R),
            p[f"l{i}_gmlp_w1"], p[f"l{i}_gmlp_b1"].reshape(1, R),
            p[f"l{i}_gmlp_w2"].reshape(1, R),
            jnp.broadcast_to(p[f"l{i}_gmlp_b2"].reshape(1, 1), (1, R)),
            p[f"l{i}_pre_w"][:R], p[f"l{i}_pre_w"][R:])
        if i == 0:
            ss, sq, mn, mx, deg = _seg_deg(bins, counts, b)
        else:
            ss, sq, mn, mx = _seg_nodeg(bins, counts, b)
        x = _post_stage(
            x, xg, a, ss, sq, mn, mx, deg, hist,
            p[f"l{i}_pre_b"].reshape(1, R), p[f"l{i}_post_w"],
            p[f"l{i}_post_b"].reshape(1, R), p[f"l{i}_lin_w"],
            p[f"l{i}_lin_b"].reshape(1, R))

    logits = _score_stage(x, p["score_w"].reshape(1, R),
                          jnp.broadcast_to(p["score_b"].reshape(1, 1), (1, R)))
    return logits[:N]
